# Initial kernel scaffold; baseline (speedup 1.0000x reference)
import jax
import jax.numpy as jnp
from jax.experimental import pallas as pl


def kernel(x, edge_index, W1, b1, W2, b2, W3, b3, W4, b4):
    raise NotImplementedError("write your pallas kernel here")


# speedup vs baseline: 11.6209x; 11.6209x over previous
import functools

import jax
import jax.numpy as jnp
from jax import lax
from jax.experimental import pallas as pl
from jax.experimental.pallas import tpu as pltpu
from jax.experimental.pallas import tpu_sc as plsc

N = 10000
E = 320000
NC = 2
NS = 16
NW = NC * NS
K = 128
CHUNKS = 79
EW = K * CHUNKS
EPAD = EW * NW
NPAD = 10240
ZROWS = NPAD // NS
NPAD1 = 10240


@functools.cache
def _make_deg():
    mesh = plsc.VectorSubcoreMesh(core_axis_name="c", subcore_axis_name="s")

    @functools.partial(
        pl.kernel,
        out_type=jax.ShapeDtypeStruct((NC * N,), jnp.float32),
        mesh=mesh,
        scratch_types=[
            pltpu.VMEM((K,), jnp.int32),
            pltpu.VMEM((K,), jnp.float32),
            pltpu.VMEM((640,), jnp.float32),
            pltpu.VMEM((N,), jnp.float32),
            pltpu.VMEM_SHARED((NPAD1,), jnp.float32),
        ],
    )
    def deg_kernel(dst_ref, out_ref, didx, ones_v, zero_v, bounce, dacc):
        c = lax.axis_index("c")
        s = lax.axis_index("s")
        one16 = jnp.ones((16,), jnp.float32)
        zero16 = jnp.zeros((16,), jnp.float32)
        for j in range(K // 16):
            ones_v[pl.ds(j * 16, 16)] = one16
        for j in range(640 // 16):
            zero_v[pl.ds(j * 16, 16)] = zero16
        pltpu.sync_copy(zero_v, dacc.at[pl.ds(pl.multiple_of(s * 640, 8), 640)])
        plsc.subcore_barrier()

        ebase = (c * NS + s) * EW

        def body(t, carry):
            off = pl.multiple_of(ebase + t * K, 8)
            pltpu.sync_copy(dst_ref.at[pl.ds(off, K)], didx)
            pltpu.sync_copy(ones_v, dacc.at[didx], add=True)
            return carry

        lax.fori_loop(0, CHUNKS, body, 0)
        plsc.subcore_barrier()

        @pl.when(s == 0)
        def _():
            pltpu.sync_copy(dacc.at[pl.ds(0, N)], bounce)
            pltpu.sync_copy(bounce,
                            out_ref.at[pl.ds(pl.multiple_of(c * N, 8), N)])

    return deg_kernel


@functools.cache
def _make_agg(C):
    mesh = plsc.VectorSubcoreMesh(core_axis_name="c", subcore_axis_name="s")

    @functools.partial(
        pl.kernel,
        out_type=jax.ShapeDtypeStruct((NC * N, C), jnp.float32),
        mesh=mesh,
        compiler_params=pltpu.CompilerParams(use_tc_tiling_on_sc=False),
        scratch_types=[
            pltpu.VMEM((K,), jnp.int32),
            pltpu.VMEM((K,), jnp.int32),
            pltpu.VMEM((K, C), jnp.float32),
            pltpu.VMEM_SHARED((NPAD, C), jnp.float32),
            pltpu.SemaphoreType.DMA,
        ],
    )
    def agg_kernel(src_ref, dst_ref, hn_ref, out_ref, sidx, didx, rows, acc,
                   sem):
        c = lax.axis_index("c")
        s = lax.axis_index("s")
        zero16 = jnp.zeros((16,), jnp.float32)

        def zrow(r, carry):
            for j in range(C // 16):
                rows[r, pl.ds(j * 16, 16)] = zero16
            return carry

        lax.fori_loop(0, K, zrow, 0)
        zb = pl.multiple_of(s * ZROWS, 8)
        for i in range(ZROWS // K):
            pltpu.sync_copy(rows, acc.at[pl.ds(zb + i * K, K)])
        plsc.subcore_barrier()

        ebase = (c * NS + s) * EW

        def body(t, carry):
            off = pl.multiple_of(ebase + t * K, 8)
            pltpu.sync_copy(src_ref.at[pl.ds(off, K)], sidx)
            pltpu.sync_copy(dst_ref.at[pl.ds(off, K)], didx)
            pltpu.async_copy(hn_ref.at[sidx], rows, sem).wait()
            pltpu.sync_copy(rows, acc.at[didx], add=True)
            return carry

        lax.fori_loop(0, CHUNKS, body, 0)
        plsc.subcore_barrier()

        wb = pl.multiple_of(s * ZROWS, 8)
        ob = pl.multiple_of(c * N, 8)

        @pl.when(s < 15)
        def _():
            pltpu.sync_copy(acc.at[pl.ds(wb, ZROWS)],
                            out_ref.at[pl.ds(ob + wb, ZROWS)])

        @pl.when(s == 15)
        def _():
            pltpu.sync_copy(acc.at[pl.ds(15 * ZROWS, N - 15 * ZROWS)],
                            out_ref.at[pl.ds(ob + 15 * ZROWS,
                                             N - 15 * ZROWS)])

    return agg_kernel


def _tc1_body(x_ref, w_ref, d0_ref, d1_ref, linn_ref, dinv_ref):
    deg = d0_ref[...] + d1_ref[...] + 1.0
    dinv = lax.rsqrt(deg)
    lin = jnp.dot(x_ref[...], w_ref[...], preferred_element_type=jnp.float32)
    linn_ref[...] = lin * dinv
    dinv_ref[...] = dinv


def _tc2_body(a0_ref, a1_ref, hn_ref, dinv_ref, b_ref, o_ref):
    dinv = dinv_ref[...]
    g = dinv * (a0_ref[...] + a1_ref[...] + hn_ref[...])
    h = g + b_ref[0:1, :]
    h = jnp.where(h > 0, h, 0.01 * h)
    o_ref[...] = h * dinv


def _tc34_body(a0_ref, a1_ref, hn_ref, dinv_ref, w_ref, b_ref, o_ref):
    dinv = dinv_ref[...]
    g = dinv * (a0_ref[...] + a1_ref[...] + hn_ref[...])
    h = jnp.dot(g, w_ref[...], preferred_element_type=jnp.float32)
    h = h + b_ref[0:1, :]
    h = jnp.where(h > 0, h, 0.01 * h)
    o_ref[...] = h * dinv


def _tc5_body(a0_ref, a1_ref, hn_ref, dinv_ref, w_ref, b_ref, o_ref):
    dinv = dinv_ref[...]
    g = dinv * (a0_ref[...] + a1_ref[...] + hn_ref[...])
    h = jnp.dot(g, w_ref[...], preferred_element_type=jnp.float32)
    o_ref[...] = h + b_ref[0:1, :]


def _f32(*shape):
    return jax.ShapeDtypeStruct(shape, jnp.float32)


def kernel(x, edge_index, W1, b1, W2, b2, W3, b3, W4, b4):
    src = edge_index[0].astype(jnp.int32)
    dst = edge_index[1].astype(jnp.int32)
    pad = EPAD - E
    srcp = jnp.concatenate([src, jnp.zeros((pad,), jnp.int32)])
    dstp = jnp.concatenate([dst, jnp.full((pad,), N, jnp.int32)])

    degp = _make_deg()(dstp)
    d0 = degp[:N, None]
    d1 = degp[N:, None]

    lin1n, dinv = pl.pallas_call(
        _tc1_body, out_shape=(_f32(N, 32), _f32(N, 1)))(x, W1, d0, d1)

    a = _make_agg(32)(srcp, dstp, lin1n)
    h1n = pl.pallas_call(_tc2_body, out_shape=_f32(N, 32))(
        a[:N], a[N:], lin1n, dinv, jnp.broadcast_to(b1, (8, 32)))

    a = _make_agg(32)(srcp, dstp, h1n)
    h2n = pl.pallas_call(_tc34_body, out_shape=_f32(N, 64))(
        a[:N], a[N:], h1n, dinv, W2, jnp.broadcast_to(b2, (8, 64)))

    a = _make_agg(64)(srcp, dstp, h2n)
    h3n = pl.pallas_call(_tc34_body, out_shape=_f32(N, 128))(
        a[:N], a[N:], h2n, dinv, W3, jnp.broadcast_to(b3, (8, 128)))

    a = _make_agg(128)(srcp, dstp, h3n)
    out = pl.pallas_call(_tc5_body, out_shape=_f32(N, 128))(
        a[:N], a[N:], h3n, dinv, W4, jnp.broadcast_to(b4, (8, 128)))
    return out


# speedup vs baseline: 12.9355x; 1.1131x over previous
import functools

import jax
import jax.numpy as jnp
from jax import lax
from jax.experimental import pallas as pl
from jax.experimental.pallas import tpu as pltpu
from jax.experimental.pallas import tpu_sc as plsc

N = 10000
E = 320000
NC = 2
NS = 16
NW = NC * NS
K = 128
CHUNKS = 80
EW = K * CHUNKS
EPAD = EW * NW
EROWS = EPAD // K
NPAD = 10240
ZROWS = NPAD // NS

_SC_PARAMS = pltpu.CompilerParams(use_tc_tiling_on_sc=False)


@functools.cache
def _make_deg():
    mesh = plsc.VectorSubcoreMesh(core_axis_name="c", subcore_axis_name="s")

    @functools.partial(
        pl.kernel,
        out_type=jax.ShapeDtypeStruct((NC * N,), jnp.float32),
        mesh=mesh,
        compiler_params=_SC_PARAMS,
        scratch_types=[
            pltpu.VMEM((CHUNKS, K), jnp.int32),
            pltpu.VMEM((K,), jnp.float32),
            pltpu.VMEM((640,), jnp.float32),
            pltpu.VMEM((N,), jnp.float32),
            pltpu.VMEM_SHARED((NPAD,), jnp.float32),
            pltpu.SemaphoreType.DMA,
        ],
    )
    def deg_kernel(dst_ref, out_ref, didx, ones_v, zero_v, bounce, dacc, sem):
        c = lax.axis_index("c")
        s = lax.axis_index("s")
        wid = c * NS + s
        one16 = jnp.ones((16,), jnp.float32)
        zero16 = jnp.zeros((16,), jnp.float32)
        for j in range(K // 16):
            ones_v[pl.ds(j * 16, 16)] = one16
        for j in range(640 // 16):
            zero_v[pl.ds(j * 16, 16)] = zero16
        rbase = pl.multiple_of(wid * CHUNKS, 8)
        pltpu.sync_copy(dst_ref.at[pl.ds(rbase, CHUNKS)], didx)
        pltpu.sync_copy(zero_v, dacc.at[pl.ds(pl.multiple_of(s * 640, 8), 640)])
        plsc.subcore_barrier()

        def fire(t, carry):
            pltpu.async_copy(ones_v, dacc.at[didx.at[t]], sem, add=True)
            return carry

        lax.fori_loop(0, CHUNKS, fire, 0)

        def drain(t, carry):
            pltpu.make_async_copy(ones_v, dacc.at[didx.at[0]], sem).wait()
            return carry

        lax.fori_loop(0, CHUNKS, drain, 0)
        plsc.subcore_barrier()

        @pl.when(s == 0)
        def _():
            pltpu.sync_copy(dacc.at[pl.ds(0, N)], bounce)
            pltpu.sync_copy(bounce,
                            out_ref.at[pl.ds(pl.multiple_of(c * N, 8), N)])

    return deg_kernel


@functools.cache
def _make_agg(C):
    SLOTS = 8
    AHEAD = 4
    G = CHUNKS // SLOTS
    assert CHUNKS % SLOTS == 0
    mesh = plsc.VectorSubcoreMesh(core_axis_name="c", subcore_axis_name="s")
    scratch = (
        [pltpu.VMEM((CHUNKS, K), jnp.int32),
         pltpu.VMEM((CHUNKS, K), jnp.int32)]
        + [pltpu.VMEM((K, C), jnp.float32) for _ in range(SLOTS)]
        + [pltpu.SemaphoreType.DMA for _ in range(2 * SLOTS)]
    )

    @functools.partial(
        pl.kernel,
        out_type=jax.ShapeDtypeStruct((NC * N, C), jnp.float32),
        mesh=mesh,
        compiler_params=_SC_PARAMS,
        scratch_types=scratch + [pltpu.VMEM_SHARED((NPAD, C), jnp.float32)],
    )
    def agg_kernel(src_ref, dst_ref, hn_ref, out_ref, sidx, didx, *rest):
        rows = rest[:SLOTS]
        gsem = rest[SLOTS:2 * SLOTS]
        ssem = rest[2 * SLOTS:3 * SLOTS]
        acc = rest[3 * SLOTS]
        c = lax.axis_index("c")
        s = lax.axis_index("s")
        wid = c * NS + s
        rbase = pl.multiple_of(wid * CHUNKS, 8)
        pltpu.sync_copy(src_ref.at[pl.ds(rbase, CHUNKS)], sidx)
        pltpu.sync_copy(dst_ref.at[pl.ds(rbase, CHUNKS)], didx)

        zero16 = jnp.zeros((16,), jnp.float32)

        def zrow(r, carry):
            for j in range(C // 16):
                rows[0][r, pl.ds(j * 16, 16)] = zero16
            return carry

        lax.fori_loop(0, K, zrow, 0)
        zb = pl.multiple_of(s * ZROWS, 8)
        for i in range(ZROWS // K):
            pltpu.sync_copy(rows[0], acc.at[pl.ds(zb + i * K, K)])
        plsc.subcore_barrier()

        def gather(t, u):
            pltpu.async_copy(hn_ref.at[sidx.at[t]], rows[u], gsem[u])

        def wait_gather(u):
            pltpu.make_async_copy(hn_ref.at[sidx.at[0]], rows[u],
                                  gsem[u]).wait()

        def scatter(t, u):
            pltpu.async_copy(rows[u], acc.at[didx.at[t]], ssem[u], add=True)

        def wait_scatter(u):
            pltpu.make_async_copy(rows[u], acc.at[didx.at[0]],
                                  ssem[u]).wait()

        for b in range(AHEAD):
            gather(b, b)

        def group(g, carry):
            t0 = g * SLOTS
            for u in range(SLOTS):
                t = t0 + u
                wait_gather(u)
                scatter(t, u)
                su = (u + AHEAD) % SLOTS
                tg = t + AHEAD
                if u >= SLOTS - AHEAD:
                    wait_scatter(su)

                    @pl.when(g < G - 1)
                    def _():
                        gather(tg, su)
                else:
                    @pl.when(g > 0)
                    def _():
                        wait_scatter(su)

                    gather(tg, su)
            return carry

        lax.fori_loop(0, G, group, 0)
        for u in range(AHEAD, SLOTS):
            wait_scatter(u)
        plsc.subcore_barrier()

        wb = pl.multiple_of(s * ZROWS, 8)
        ob = pl.multiple_of(c * N, 8)

        @pl.when(s < 15)
        def _():
            pltpu.sync_copy(acc.at[pl.ds(wb, ZROWS)],
                            out_ref.at[pl.ds(ob + wb, ZROWS)])

        @pl.when(s == 15)
        def _():
            pltpu.sync_copy(acc.at[pl.ds(15 * ZROWS, N - 15 * ZROWS)],
                            out_ref.at[pl.ds(ob + 15 * ZROWS,
                                             N - 15 * ZROWS)])

    return agg_kernel


def _tc1_body(x_ref, w_ref, d0_ref, d1_ref, linn_ref, dinv_ref):
    deg = d0_ref[...] + d1_ref[...] + 1.0
    dinv = lax.rsqrt(deg)
    lin = jnp.dot(x_ref[...], w_ref[...], preferred_element_type=jnp.float32)
    linn_ref[...] = lin * dinv
    dinv_ref[...] = dinv


def _tc2_body(a0_ref, a1_ref, hn_ref, dinv_ref, b_ref, o_ref):
    dinv = dinv_ref[...]
    g = dinv * (a0_ref[...] + a1_ref[...] + hn_ref[...])
    h = g + b_ref[0:1, :]
    h = jnp.where(h > 0, h, 0.01 * h)
    o_ref[...] = h * dinv


def _tc34_body(a0_ref, a1_ref, hn_ref, dinv_ref, w_ref, b_ref, o_ref):
    dinv = dinv_ref[...]
    g = dinv * (a0_ref[...] + a1_ref[...] + hn_ref[...])
    h = jnp.dot(g, w_ref[...], preferred_element_type=jnp.float32)
    h = h + b_ref[0:1, :]
    h = jnp.where(h > 0, h, 0.01 * h)
    o_ref[...] = h * dinv


def _tc4_body(a0_ref, a1_ref, hn_ref, dinv_ref, w_ref, b_ref, oL_ref, oR_ref):
    dinv = dinv_ref[...]
    g = dinv * (a0_ref[...] + a1_ref[...] + hn_ref[...])
    h = jnp.dot(g, w_ref[...], preferred_element_type=jnp.float32)
    h = h + b_ref[0:1, :]
    h = jnp.where(h > 0, h, 0.01 * h)
    h = h * dinv
    oL_ref[...] = h[:, :64]
    oR_ref[...] = h[:, 64:]


def _tc5_body(aL0_ref, aL1_ref, aR0_ref, aR1_ref, hL_ref, hR_ref,
              dinv_ref, w_ref, b_ref, o_ref):
    dinv = dinv_ref[...]
    gL = dinv * (aL0_ref[...] + aL1_ref[...] + hL_ref[...])
    gR = dinv * (aR0_ref[...] + aR1_ref[...] + hR_ref[...])
    w = w_ref[...]
    h = jnp.dot(gL, w[:64, :], preferred_element_type=jnp.float32)
    h = h + jnp.dot(gR, w[64:, :], preferred_element_type=jnp.float32)
    o_ref[...] = h + b_ref[0:1, :]


def _f32(*shape):
    return jax.ShapeDtypeStruct(shape, jnp.float32)


def kernel(x, edge_index, W1, b1, W2, b2, W3, b3, W4, b4):
    src = edge_index[0].astype(jnp.int32)
    dst = edge_index[1].astype(jnp.int32)
    pad = EPAD - E
    srcp = jnp.concatenate([src, jnp.zeros((pad,), jnp.int32)])
    dstp = jnp.concatenate([dst, jnp.full((pad,), N, jnp.int32)])
    srcp = srcp.reshape(EROWS, K)
    dstp = dstp.reshape(EROWS, K)

    degp = _make_deg()(dstp)
    d0 = degp[:N, None]
    d1 = degp[N:, None]

    lin1n, dinv = pl.pallas_call(
        _tc1_body, out_shape=(_f32(N, 32), _f32(N, 1)))(x, W1, d0, d1)

    a = _make_agg(32)(srcp, dstp, lin1n)
    h1n = pl.pallas_call(_tc2_body, out_shape=_f32(N, 32))(
        a[:N], a[N:], lin1n, dinv, jnp.broadcast_to(b1, (8, 32)))

    a = _make_agg(32)(srcp, dstp, h1n)
    h2n = pl.pallas_call(_tc34_body, out_shape=_f32(N, 64))(
        a[:N], a[N:], h1n, dinv, W2, jnp.broadcast_to(b2, (8, 64)))

    a = _make_agg(64)(srcp, dstp, h2n)
    h3nL, h3nR = pl.pallas_call(
        _tc4_body, out_shape=(_f32(N, 64), _f32(N, 64)))(
        a[:N], a[N:], h2n, dinv, W3, jnp.broadcast_to(b3, (8, 128)))

    aL = _make_agg(64)(srcp, dstp, h3nL)
    aR = _make_agg(64)(srcp, dstp, h3nR)
    out = pl.pallas_call(_tc5_body, out_shape=_f32(N, 128))(
        aL[:N], aL[N:], aR[:N], aR[N:], h3nL, h3nR, dinv, W4,
        jnp.broadcast_to(b4, (8, 128)))
    return out


# speedup vs baseline: 32.4955x; 2.5121x over previous
import functools

import jax
import jax.numpy as jnp
from jax import lax
from jax.experimental import pallas as pl
from jax.experimental.pallas import tpu as pltpu
from jax.experimental.pallas import tpu_sc as plsc

N = 10000
E = 320000
NC = 2
NS = 16
NW = NC * NS
K = 128
CHUNKS = 80
EW = K * CHUNKS
EPAD = EW * NW
EROWS = EPAD // K
NPAD = 10240
ZROWS = NPAD // NS

_SC_PARAMS = pltpu.CompilerParams(use_tc_tiling_on_sc=False)


@functools.cache
def _make_deg():
    mesh = plsc.VectorSubcoreMesh(core_axis_name="c", subcore_axis_name="s")

    @functools.partial(
        pl.kernel,
        out_type=jax.ShapeDtypeStruct((NC * N,), jnp.float32),
        mesh=mesh,
        compiler_params=_SC_PARAMS,
        scratch_types=[
            pltpu.VMEM((CHUNKS, K), jnp.int32),
            pltpu.VMEM((K,), jnp.float32),
            pltpu.VMEM((640,), jnp.float32),
            pltpu.VMEM((N,), jnp.float32),
            pltpu.VMEM_SHARED((NPAD,), jnp.float32),
            pltpu.SemaphoreType.DMA,
        ],
    )
    def deg_kernel(dst_ref, out_ref, didx, ones_v, zero_v, bounce, dacc, sem):
        c = lax.axis_index("c")
        s = lax.axis_index("s")
        wid = c * NS + s
        one16 = jnp.ones((16,), jnp.float32)
        zero16 = jnp.zeros((16,), jnp.float32)
        for j in range(K // 16):
            ones_v[pl.ds(j * 16, 16)] = one16
        for j in range(640 // 16):
            zero_v[pl.ds(j * 16, 16)] = zero16
        rbase = pl.multiple_of(wid * CHUNKS, 8)
        pltpu.sync_copy(dst_ref.at[pl.ds(rbase, CHUNKS)], didx)
        pltpu.sync_copy(zero_v, dacc.at[pl.ds(pl.multiple_of(s * 640, 8), 640)])
        plsc.subcore_barrier()

        def fire(t, carry):
            pltpu.async_copy(ones_v, dacc.at[didx.at[t]], sem, add=True)
            return carry

        lax.fori_loop(0, CHUNKS, fire, 0)

        def drain(t, carry):
            pltpu.make_async_copy(ones_v, dacc.at[didx.at[0]], sem).wait()
            return carry

        lax.fori_loop(0, CHUNKS, drain, 0)
        plsc.subcore_barrier()

        @pl.when(s == 0)
        def _():
            pltpu.sync_copy(dacc.at[pl.ds(0, N)], bounce)
            pltpu.sync_copy(bounce,
                            out_ref.at[pl.ds(pl.multiple_of(c * N, 8), N)])

    return deg_kernel


@functools.cache
def _make_agg(C):
    SLOTS = 8
    AHEAD = 4
    G = CHUNKS // SLOTS
    assert CHUNKS % SLOTS == 0
    mesh = plsc.VectorSubcoreMesh(core_axis_name="c", subcore_axis_name="s")
    scratch = (
        [pltpu.VMEM((CHUNKS, K), jnp.int32),
         pltpu.VMEM((CHUNKS, K), jnp.int32)]
        + [pltpu.VMEM((K, C), jnp.float32) for _ in range(SLOTS)]
        + [pltpu.SemaphoreType.DMA for _ in range(2 * SLOTS)]
    )

    @functools.partial(
        pl.kernel,
        out_type=jax.ShapeDtypeStruct((NC * N, C), jnp.float32),
        mesh=mesh,
        compiler_params=_SC_PARAMS,
        scratch_types=scratch + [pltpu.VMEM_SHARED((NPAD, C), jnp.float32)],
    )
    def agg_kernel(src_ref, dst_ref, hn_ref, out_ref, sidx, didx, *rest):
        rows = rest[:SLOTS]
        gsem = rest[SLOTS:2 * SLOTS]
        ssem = rest[2 * SLOTS:3 * SLOTS]
        acc = rest[3 * SLOTS]
        c = lax.axis_index("c")
        s = lax.axis_index("s")
        wid = c * NS + s
        rbase = pl.multiple_of(wid * CHUNKS, 8)
        pltpu.sync_copy(src_ref.at[pl.ds(rbase, CHUNKS)], sidx)
        pltpu.sync_copy(dst_ref.at[pl.ds(rbase, CHUNKS)], didx)

        zero16 = jnp.zeros((16,), jnp.float32)

        def zrow(r, carry):
            for j in range(C // 16):
                rows[0][r, pl.ds(j * 16, 16)] = zero16
            return carry

        lax.fori_loop(0, K, zrow, 0)
        zb = pl.multiple_of(s * ZROWS, 8)
        for i in range(ZROWS // K):
            pltpu.sync_copy(rows[0], acc.at[pl.ds(zb + i * K, K)])
        plsc.subcore_barrier()

        def gather(t, u):
            pltpu.async_copy(hn_ref.at[sidx.at[t]], rows[u], gsem[u])

        def wait_gather(u):
            pltpu.make_async_copy(hn_ref.at[sidx.at[0]], rows[u],
                                  gsem[u]).wait()

        def scatter(t, u):
            pltpu.async_copy(rows[u], acc.at[didx.at[t]], ssem[u], add=True)

        def wait_scatter(u):
            pltpu.make_async_copy(rows[u], acc.at[didx.at[0]],
                                  ssem[u]).wait()

        for b in range(AHEAD):
            gather(b, b)

        def group(g, carry):
            t0 = g * SLOTS
            for u in range(SLOTS):
                t = t0 + u
                wait_gather(u)
                scatter(t, u)
                su = (u + AHEAD) % SLOTS
                tg = t + AHEAD
                if u >= SLOTS - AHEAD:
                    wait_scatter(su)

                    @pl.when(g < G - 1)
                    def _():
                        gather(tg, su)
                else:
                    @pl.when(g > 0)
                    def _():
                        wait_scatter(su)

                    gather(tg, su)
            return carry

        lax.fori_loop(0, G, group, 0)
        for u in range(AHEAD, SLOTS):
            wait_scatter(u)
        plsc.subcore_barrier()

        wb = pl.multiple_of(s * ZROWS, 8)
        ob = pl.multiple_of(c * N, 8)

        @pl.when(s < 15)
        def _():
            pltpu.sync_copy(acc.at[pl.ds(wb, ZROWS)],
                            out_ref.at[pl.ds(ob + wb, ZROWS)])

        @pl.when(s == 15)
        def _():
            pltpu.sync_copy(acc.at[pl.ds(15 * ZROWS, N - 15 * ZROWS)],
                            out_ref.at[pl.ds(ob + 15 * ZROWS,
                                             N - 15 * ZROWS)])

    return agg_kernel


def _tc1_body(x_ref, w_ref, d0_ref, d1_ref, linn_ref, dinv_ref):
    deg = d0_ref[...] + d1_ref[...] + 1.0
    dinv = lax.rsqrt(deg)
    lin = jnp.dot(x_ref[...], w_ref[...], preferred_element_type=jnp.float32)
    linn_ref[...] = lin * dinv
    dinv_ref[...] = dinv


def _tc2_body(a0_ref, a1_ref, hn_ref, dinv_ref, b_ref, o_ref):
    dinv = dinv_ref[...]
    g = dinv * (a0_ref[...] + a1_ref[...] + hn_ref[...])
    h = g + b_ref[0:1, :]
    h = jnp.where(h > 0, h, 0.01 * h)
    o_ref[...] = h * dinv


def _tc34_body(a0_ref, a1_ref, hn_ref, dinv_ref, w_ref, b_ref, o_ref):
    dinv = dinv_ref[...]
    g = dinv * (a0_ref[...] + a1_ref[...] + hn_ref[...])
    h = jnp.dot(g, w_ref[...], preferred_element_type=jnp.float32)
    h = h + b_ref[0:1, :]
    h = jnp.where(h > 0, h, 0.01 * h)
    o_ref[...] = h * dinv


def _tc4_body(a0_ref, a1_ref, hn_ref, dinv_ref, w_ref, b_ref, oL_ref, oR_ref):
    dinv = dinv_ref[...]
    g = dinv * (a0_ref[...] + a1_ref[...] + hn_ref[...])
    h = jnp.dot(g, w_ref[...], preferred_element_type=jnp.float32)
    h = h + b_ref[0:1, :]
    h = jnp.where(h > 0, h, 0.01 * h)
    h = h * dinv
    oL_ref[...] = h[:, :64]
    oR_ref[...] = h[:, 64:]


def _tc5_body(aL0_ref, aL1_ref, aR0_ref, aR1_ref, hL_ref, hR_ref,
              dinv_ref, w_ref, b_ref, o_ref):
    dinv = dinv_ref[...]
    gL = dinv * (aL0_ref[...] + aL1_ref[...] + hL_ref[...])
    gR = dinv * (aR0_ref[...] + aR1_ref[...] + hR_ref[...])
    w = w_ref[...]
    h = jnp.dot(gL, w[:64, :], preferred_element_type=jnp.float32)
    h = h + jnp.dot(gR, w[64:, :], preferred_element_type=jnp.float32)
    o_ref[...] = h + b_ref[0:1, :]


def _f32(*shape):
    return jax.ShapeDtypeStruct(shape, jnp.float32)


def kernel(x, edge_index, W1, b1, W2, b2, W3, b3, W4, b4):
    src = edge_index[0].astype(jnp.int32)
    dst = edge_index[1].astype(jnp.int32)
    pad = EPAD - E
    pidx = jnp.arange(pad, dtype=jnp.int32)
    srcp = jnp.concatenate([src, pidx % N])
    dstp = jnp.concatenate([dst, N + pidx % (NPAD - N)])
    srcp = srcp.reshape(EROWS, K)
    dstp = dstp.reshape(EROWS, K)

    degp = _make_deg()(dstp)
    d0 = degp[:N, None]
    d1 = degp[N:, None]

    lin1n, dinv = pl.pallas_call(
        _tc1_body, out_shape=(_f32(N, 32), _f32(N, 1)))(x, W1, d0, d1)

    a = _make_agg(32)(srcp, dstp, lin1n)
    h1n = pl.pallas_call(_tc2_body, out_shape=_f32(N, 32))(
        a[:N], a[N:], lin1n, dinv, jnp.broadcast_to(b1, (8, 32)))

    a = _make_agg(32)(srcp, dstp, h1n)
    h2n = pl.pallas_call(_tc34_body, out_shape=_f32(N, 64))(
        a[:N], a[N:], h1n, dinv, W2, jnp.broadcast_to(b2, (8, 64)))

    a = _make_agg(64)(srcp, dstp, h2n)
    h3nL, h3nR = pl.pallas_call(
        _tc4_body, out_shape=(_f32(N, 64), _f32(N, 64)))(
        a[:N], a[N:], h2n, dinv, W3, jnp.broadcast_to(b3, (8, 128)))

    aL = _make_agg(64)(srcp, dstp, h3nL)
    aR = _make_agg(64)(srcp, dstp, h3nR)
    out = pl.pallas_call(_tc5_body, out_shape=_f32(N, 128))(
        aL[:N], aL[N:], aR[:N], aR[N:], h3nL, h3nR, dinv, W4,
        jnp.broadcast_to(b4, (8, 128)))
    return out


# speedup vs baseline: 36.2475x; 1.1155x over previous
import functools

import numpy as np

import jax
import jax.numpy as jnp
from jax import lax
from jax.experimental import pallas as pl
from jax.experimental.pallas import tpu as pltpu
from jax.experimental.pallas import tpu_sc as plsc

N = 10000
E = 320000
NC = 2
NS = 16
NW = NC * NS
K = 128
CHUNKS = 80
EW = K * CHUNKS
EPAD = EW * NW
EROWS = EPAD // K
NPAD = 10240
ZROWS = NPAD // NS

_SC_PARAMS = pltpu.CompilerParams(use_tc_tiling_on_sc=False)


@functools.cache
def _make_deg():
    mesh = plsc.VectorSubcoreMesh(core_axis_name="c", subcore_axis_name="s")

    @functools.partial(
        pl.kernel,
        out_type=jax.ShapeDtypeStruct((NC, N), jnp.float32),
        mesh=mesh,
        compiler_params=_SC_PARAMS,
        scratch_types=[
            pltpu.VMEM((CHUNKS, K), jnp.int32),
            pltpu.VMEM((K,), jnp.float32),
            pltpu.VMEM((640,), jnp.float32),
            pltpu.VMEM((N,), jnp.float32),
            pltpu.VMEM_SHARED((NPAD,), jnp.float32),
            pltpu.SemaphoreType.DMA,
        ],
    )
    def deg_kernel(dst_ref, out_ref, didx, ones_v, zero_v, bounce, dacc, sem):
        c = lax.axis_index("c")
        s = lax.axis_index("s")
        wid = c * NS + s
        one16 = jnp.ones((16,), jnp.float32)
        zero16 = jnp.zeros((16,), jnp.float32)
        for j in range(K // 16):
            ones_v[pl.ds(j * 16, 16)] = one16
        for j in range(640 // 16):
            zero_v[pl.ds(j * 16, 16)] = zero16
        rbase = pl.multiple_of(wid * CHUNKS, 8)
        pltpu.sync_copy(dst_ref.at[pl.ds(rbase, CHUNKS)], didx)
        pltpu.sync_copy(zero_v, dacc.at[pl.ds(pl.multiple_of(s * 640, 8), 640)])
        plsc.subcore_barrier()

        def fire(t, carry):
            pltpu.async_copy(ones_v, dacc.at[didx.at[t]], sem, add=True)
            return carry

        lax.fori_loop(0, CHUNKS, fire, 0)

        def drain(t, carry):
            pltpu.make_async_copy(ones_v, dacc.at[didx.at[0]], sem).wait()
            return carry

        lax.fori_loop(0, CHUNKS, drain, 0)
        plsc.subcore_barrier()

        @pl.when(s == 0)
        def _():
            pltpu.sync_copy(dacc.at[pl.ds(0, N)], bounce)
            pltpu.sync_copy(bounce, out_ref.at[c])

    return deg_kernel


@functools.cache
def _make_agg(C):
    SLOTS = 8
    AHEAD = 4
    G = CHUNKS // SLOTS
    assert CHUNKS % SLOTS == 0
    mesh = plsc.VectorSubcoreMesh(core_axis_name="c", subcore_axis_name="s")
    scratch = (
        [pltpu.VMEM((CHUNKS, K), jnp.int32),
         pltpu.VMEM((CHUNKS, K), jnp.int32)]
        + [pltpu.VMEM((K, C), jnp.float32) for _ in range(SLOTS)]
        + [pltpu.SemaphoreType.DMA for _ in range(2 * SLOTS)]
    )

    @functools.partial(
        pl.kernel,
        out_type=jax.ShapeDtypeStruct((NC * N, C), jnp.float32),
        mesh=mesh,
        compiler_params=_SC_PARAMS,
        scratch_types=scratch + [pltpu.VMEM_SHARED((NPAD, C), jnp.float32)],
    )
    def agg_kernel(src_ref, dst_ref, hn_ref, out_ref, sidx, didx, *rest):
        rows = rest[:SLOTS]
        gsem = rest[SLOTS:2 * SLOTS]
        ssem = rest[2 * SLOTS:3 * SLOTS]
        acc = rest[3 * SLOTS]
        c = lax.axis_index("c")
        s = lax.axis_index("s")
        wid = c * NS + s
        rbase = pl.multiple_of(wid * CHUNKS, 8)
        pltpu.sync_copy(src_ref.at[pl.ds(rbase, CHUNKS)], sidx)
        pltpu.sync_copy(dst_ref.at[pl.ds(rbase, CHUNKS)], didx)

        zero16 = jnp.zeros((16,), jnp.float32)

        def zrow(r, carry):
            for j in range(C // 16):
                rows[0][r, pl.ds(j * 16, 16)] = zero16
            return carry

        lax.fori_loop(0, K, zrow, 0)
        zb = pl.multiple_of(s * ZROWS, 8)
        for i in range(ZROWS // K):
            pltpu.sync_copy(rows[0], acc.at[pl.ds(zb + i * K, K)])
        plsc.subcore_barrier()

        def gather(t, u):
            pltpu.async_copy(hn_ref.at[sidx.at[t]], rows[u], gsem[u])

        def wait_gather(u):
            pltpu.make_async_copy(hn_ref.at[sidx.at[0]], rows[u],
                                  gsem[u]).wait()

        def scatter(t, u):
            pltpu.async_copy(rows[u], acc.at[didx.at[t]], ssem[u], add=True)

        def wait_scatter(u):
            pltpu.make_async_copy(rows[u], acc.at[didx.at[0]],
                                  ssem[u]).wait()

        for b in range(AHEAD):
            gather(b, b)

        def group(g, carry):
            t0 = g * SLOTS
            for u in range(SLOTS):
                t = t0 + u
                wait_gather(u)
                scatter(t, u)
                su = (u + AHEAD) % SLOTS
                tg = t + AHEAD
                if u >= SLOTS - AHEAD:
                    wait_scatter(su)

                    @pl.when(g < G - 1)
                    def _():
                        gather(tg, su)
                else:
                    @pl.when(g > 0)
                    def _():
                        wait_scatter(su)

                    gather(tg, su)
            return carry

        lax.fori_loop(0, G, group, 0)
        for u in range(AHEAD, SLOTS):
            wait_scatter(u)
        plsc.subcore_barrier()

        wb = pl.multiple_of(s * ZROWS, 8)
        ob = pl.multiple_of(c * N, 8)

        @pl.when(s < 15)
        def _():
            pltpu.sync_copy(acc.at[pl.ds(wb, ZROWS)],
                            out_ref.at[pl.ds(ob + wb, ZROWS)])

        @pl.when(s == 15)
        def _():
            pltpu.sync_copy(acc.at[pl.ds(15 * ZROWS, N - 15 * ZROWS)],
                            out_ref.at[pl.ds(ob + 15 * ZROWS,
                                             N - 15 * ZROWS)])

    return agg_kernel


def _tc1_body(x_ref, w_ref, deg_ref, linn_ref, dinv_ref):
    dp = deg_ref[...]
    deg = dp[0:1, :] + dp[1:2, :] + 1.0
    dinv = jnp.reshape(lax.rsqrt(deg), (N, 1))
    lin = jnp.dot(x_ref[...], w_ref[...], preferred_element_type=jnp.float32)
    linn_ref[...] = lin * dinv
    dinv_ref[...] = dinv


def _tc2_body(a_ref, hn_ref, dinv_ref, b_ref, o_ref):
    dinv = dinv_ref[...]
    g = dinv * (a_ref[0:N, :] + a_ref[N:, :] + hn_ref[...])
    h = g + b_ref[0:1, :]
    h = jnp.where(h > 0, h, 0.01 * h)
    o_ref[...] = h * dinv


def _tc34_body(a_ref, hn_ref, dinv_ref, w_ref, b_ref, o_ref):
    dinv = dinv_ref[...]
    g = dinv * (a_ref[0:N, :] + a_ref[N:, :] + hn_ref[...])
    h = jnp.dot(g, w_ref[...], preferred_element_type=jnp.float32)
    h = h + b_ref[0:1, :]
    h = jnp.where(h > 0, h, 0.01 * h)
    o_ref[...] = h * dinv


def _tc4_body(a_ref, hn_ref, dinv_ref, w_ref, b_ref, oL_ref, oR_ref):
    dinv = dinv_ref[...]
    g = dinv * (a_ref[0:N, :] + a_ref[N:, :] + hn_ref[...])
    h = jnp.dot(g, w_ref[...], preferred_element_type=jnp.float32)
    h = h + b_ref[0:1, :]
    h = jnp.where(h > 0, h, 0.01 * h)
    h = h * dinv
    oL_ref[...] = h[:, :64]
    oR_ref[...] = h[:, 64:]


def _tc5_body(aL_ref, aR_ref, hL_ref, hR_ref, dinv_ref, w_ref, b_ref, o_ref):
    dinv = dinv_ref[...]
    gL = dinv * (aL_ref[0:N, :] + aL_ref[N:, :] + hL_ref[...])
    gR = dinv * (aR_ref[0:N, :] + aR_ref[N:, :] + hR_ref[...])
    w = w_ref[...]
    h = jnp.dot(gL, w[:64, :], preferred_element_type=jnp.float32)
    h = h + jnp.dot(gR, w[64:, :], preferred_element_type=jnp.float32)
    o_ref[...] = h + b_ref[0:1, :]


def _f32(*shape):
    return jax.ShapeDtypeStruct(shape, jnp.float32)


_PAD_SRC = (np.arange(EPAD - E) % N).astype(np.int32)
_PAD_DST = (N + np.arange(EPAD - E) % (NPAD - N)).astype(np.int32)


def kernel(x, edge_index, W1, b1, W2, b2, W3, b3, W4, b4):
    src = edge_index[0].astype(jnp.int32)
    dst = edge_index[1].astype(jnp.int32)
    srcp = jnp.concatenate([src, jnp.asarray(_PAD_SRC)]).reshape(EROWS, K)
    dstp = jnp.concatenate([dst, jnp.asarray(_PAD_DST)]).reshape(EROWS, K)

    degp = _make_deg()(dstp)

    lin1n, dinv = pl.pallas_call(
        _tc1_body, out_shape=(_f32(N, 32), _f32(N, 1)))(x, W1, degp)

    a = _make_agg(32)(srcp, dstp, lin1n)
    h1n = pl.pallas_call(_tc2_body, out_shape=_f32(N, 32))(
        a, lin1n, dinv, jnp.broadcast_to(b1, (8, 32)))

    a = _make_agg(32)(srcp, dstp, h1n)
    h2n = pl.pallas_call(_tc34_body, out_shape=_f32(N, 64))(
        a, h1n, dinv, W2, jnp.broadcast_to(b2, (8, 64)))

    a = _make_agg(64)(srcp, dstp, h2n)
    h3nL, h3nR = pl.pallas_call(
        _tc4_body, out_shape=(_f32(N, 64), _f32(N, 64)))(
        a, h2n, dinv, W3, jnp.broadcast_to(b3, (8, 128)))

    aL = _make_agg(64)(srcp, dstp, h3nL)
    aR = _make_agg(64)(srcp, dstp, h3nR)
    out = pl.pallas_call(_tc5_body, out_shape=_f32(N, 128))(
        aL, aR, h3nL, h3nR, dinv, W4, jnp.broadcast_to(b4, (8, 128)))
    return out


# speedup vs baseline: 36.6251x; 1.0104x over previous
import functools

import numpy as np

import jax
import jax.numpy as jnp
from jax import lax
from jax.experimental import pallas as pl
from jax.experimental.pallas import tpu as pltpu
from jax.experimental.pallas import tpu_sc as plsc

N = 10000
E = 320000
NC = 2
NS = 16
NW = NC * NS
K = 128
CHUNKS = 80
EROWS = E // K
LAST_REAL = EROWS - 31 * CHUNKS
PADC = 4
LAST_CHUNKS = LAST_REAL + PADC
NPAD = 10240
ZROWS = NPAD // NS

_SC_PARAMS = pltpu.CompilerParams(use_tc_tiling_on_sc=False)


@functools.cache
def _make_deg():
    mesh = plsc.VectorSubcoreMesh(core_axis_name="c", subcore_axis_name="s")

    @functools.partial(
        pl.kernel,
        out_type=jax.ShapeDtypeStruct((NC, N), jnp.float32),
        mesh=mesh,
        compiler_params=_SC_PARAMS,
        scratch_types=[
            pltpu.VMEM((CHUNKS, K), jnp.int32),
            pltpu.VMEM((K,), jnp.float32),
            pltpu.VMEM((640,), jnp.float32),
            pltpu.VMEM((N,), jnp.float32),
            pltpu.VMEM_SHARED((NPAD,), jnp.float32),
            pltpu.SemaphoreType.DMA,
        ],
    )
    def deg_kernel(ei_ref, pdst_ref, out_ref, didx, ones_v, zero_v, bounce,
                   dacc, sem):
        c = lax.axis_index("c")
        s = lax.axis_index("s")
        wid = c * NS + s
        one16 = jnp.ones((16,), jnp.float32)
        zero16 = jnp.zeros((16,), jnp.float32)
        for j in range(K // 16):
            ones_v[pl.ds(j * 16, 16)] = one16
        for j in range(640 // 16):
            zero_v[pl.ds(j * 16, 16)] = zero16

        @pl.when(wid < NW - 1)
        def _():
            pltpu.sync_copy(ei_ref.at[1, pl.ds(wid * CHUNKS, CHUNKS)], didx)

        @pl.when(wid == NW - 1)
        def _():
            pltpu.sync_copy(ei_ref.at[1, pl.ds(31 * CHUNKS, LAST_REAL)],
                            didx.at[pl.ds(0, LAST_REAL)])
            pltpu.sync_copy(pdst_ref, didx.at[pl.ds(LAST_REAL, PADC)])

        nch = jnp.where(wid < NW - 1, CHUNKS, LAST_CHUNKS)
        pltpu.sync_copy(zero_v, dacc.at[pl.ds(pl.multiple_of(s * 640, 8), 640)])
        plsc.subcore_barrier()

        def fire(t, carry):
            pltpu.async_copy(ones_v, dacc.at[didx.at[t]], sem, add=True)
            return carry

        lax.fori_loop(0, nch, fire, 0)

        def drain(t, carry):
            pltpu.make_async_copy(ones_v, dacc.at[didx.at[0]], sem).wait()
            return carry

        lax.fori_loop(0, nch, drain, 0)
        plsc.subcore_barrier()

        @pl.when(s == 0)
        def _():
            pltpu.sync_copy(dacc.at[pl.ds(0, N)], bounce)
            pltpu.sync_copy(bounce, out_ref.at[c])

    return deg_kernel


@functools.cache
def _make_agg(C):
    SLOTS = 8
    AHEAD = 4
    assert CHUNKS % SLOTS == 0 and LAST_CHUNKS % SLOTS == 0
    mesh = plsc.VectorSubcoreMesh(core_axis_name="c", subcore_axis_name="s")
    scratch = (
        [pltpu.VMEM((CHUNKS, K), jnp.int32),
         pltpu.VMEM((CHUNKS, K), jnp.int32)]
        + [pltpu.VMEM((K, C), jnp.float32) for _ in range(SLOTS)]
        + [pltpu.SemaphoreType.DMA for _ in range(2 * SLOTS)]
    )

    @functools.partial(
        pl.kernel,
        out_type=jax.ShapeDtypeStruct((NC * N, C), jnp.float32),
        mesh=mesh,
        compiler_params=_SC_PARAMS,
        scratch_types=scratch + [pltpu.VMEM_SHARED((NPAD, C), jnp.float32)],
    )
    def agg_kernel(ei_ref, psrc_ref, pdst_ref, hn_ref, out_ref, sidx, didx,
                   *rest):
        rows = rest[:SLOTS]
        gsem = rest[SLOTS:2 * SLOTS]
        ssem = rest[2 * SLOTS:3 * SLOTS]
        acc = rest[3 * SLOTS]
        c = lax.axis_index("c")
        s = lax.axis_index("s")
        wid = c * NS + s

        @pl.when(wid < NW - 1)
        def _():
            pltpu.sync_copy(ei_ref.at[0, pl.ds(wid * CHUNKS, CHUNKS)], sidx)
            pltpu.sync_copy(ei_ref.at[1, pl.ds(wid * CHUNKS, CHUNKS)], didx)

        @pl.when(wid == NW - 1)
        def _():
            pltpu.sync_copy(ei_ref.at[0, pl.ds(31 * CHUNKS, LAST_REAL)],
                            sidx.at[pl.ds(0, LAST_REAL)])
            pltpu.sync_copy(psrc_ref, sidx.at[pl.ds(LAST_REAL, PADC)])
            pltpu.sync_copy(ei_ref.at[1, pl.ds(31 * CHUNKS, LAST_REAL)],
                            didx.at[pl.ds(0, LAST_REAL)])
            pltpu.sync_copy(pdst_ref, didx.at[pl.ds(LAST_REAL, PADC)])

        ngroups = jnp.where(wid < NW - 1, CHUNKS // SLOTS,
                            LAST_CHUNKS // SLOTS)

        zero16 = jnp.zeros((16,), jnp.float32)

        def zrow(r, carry):
            for j in range(C // 16):
                rows[0][r, pl.ds(j * 16, 16)] = zero16
            return carry

        lax.fori_loop(0, K, zrow, 0)
        zb = pl.multiple_of(s * ZROWS, 8)
        for i in range(ZROWS // K):
            pltpu.sync_copy(rows[0], acc.at[pl.ds(zb + i * K, K)])
        plsc.subcore_barrier()

        def gather(t, u):
            pltpu.async_copy(hn_ref.at[sidx.at[t]], rows[u], gsem[u])

        def wait_gather(u):
            pltpu.make_async_copy(hn_ref.at[sidx.at[0]], rows[u],
                                  gsem[u]).wait()

        def scatter(t, u):
            pltpu.async_copy(rows[u], acc.at[didx.at[t]], ssem[u], add=True)

        def wait_scatter(u):
            pltpu.make_async_copy(rows[u], acc.at[didx.at[0]],
                                  ssem[u]).wait()

        for b in range(AHEAD):
            gather(b, b)

        def group(g, carry):
            t0 = g * SLOTS
            for u in range(SLOTS):
                t = t0 + u
                wait_gather(u)
                scatter(t, u)
                su = (u + AHEAD) % SLOTS
                tg = t + AHEAD
                if u >= SLOTS - AHEAD:
                    wait_scatter(su)

                    @pl.when(g < ngroups - 1)
                    def _():
                        gather(tg, su)
                else:
                    @pl.when(g > 0)
                    def _():
                        wait_scatter(su)

                    gather(tg, su)
            return carry

        lax.fori_loop(0, ngroups, group, 0)
        for u in range(AHEAD, SLOTS):
            wait_scatter(u)
        plsc.subcore_barrier()

        wb = pl.multiple_of(s * ZROWS, 8)
        ob = pl.multiple_of(c * N, 8)

        @pl.when(s < 15)
        def _():
            pltpu.sync_copy(acc.at[pl.ds(wb, ZROWS)],
                            out_ref.at[pl.ds(ob + wb, ZROWS)])

        @pl.when(s == 15)
        def _():
            pltpu.sync_copy(acc.at[pl.ds(15 * ZROWS, N - 15 * ZROWS)],
                            out_ref.at[pl.ds(ob + 15 * ZROWS,
                                             N - 15 * ZROWS)])

    return agg_kernel


def _tc1_body(x_ref, w_ref, deg_ref, linn_ref, dinv_ref):
    dp = deg_ref[...]
    deg = dp[0:1, :] + dp[1:2, :] + 1.0
    dinv = jnp.reshape(lax.rsqrt(deg), (N, 1))
    lin = jnp.dot(x_ref[...], w_ref[...], preferred_element_type=jnp.float32)
    linn_ref[...] = lin * dinv
    dinv_ref[...] = dinv


def _tc2_body(a_ref, hn_ref, dinv_ref, b_ref, o_ref):
    dinv = dinv_ref[...]
    g = dinv * (a_ref[0:N, :] + a_ref[N:, :] + hn_ref[...])
    h = g + b_ref[0:1, :]
    h = jnp.where(h > 0, h, 0.01 * h)
    o_ref[...] = h * dinv


def _tc34_body(a_ref, hn_ref, dinv_ref, w_ref, b_ref, o_ref):
    dinv = dinv_ref[...]
    g = dinv * (a_ref[0:N, :] + a_ref[N:, :] + hn_ref[...])
    h = jnp.dot(g, w_ref[...], preferred_element_type=jnp.float32)
    h = h + b_ref[0:1, :]
    h = jnp.where(h > 0, h, 0.01 * h)
    o_ref[...] = h * dinv


def _tc4_body(a_ref, hn_ref, dinv_ref, w_ref, b_ref, oL_ref, oR_ref):
    dinv = dinv_ref[...]
    g = dinv * (a_ref[0:N, :] + a_ref[N:, :] + hn_ref[...])
    h = jnp.dot(g, w_ref[...], preferred_element_type=jnp.float32)
    h = h + b_ref[0:1, :]
    h = jnp.where(h > 0, h, 0.01 * h)
    h = h * dinv
    oL_ref[...] = h[:, :64]
    oR_ref[...] = h[:, 64:]


def _tc5_body(aL_ref, aR_ref, hL_ref, hR_ref, dinv_ref, w_ref, b_ref, o_ref):
    dinv = dinv_ref[...]
    gL = dinv * (aL_ref[0:N, :] + aL_ref[N:, :] + hL_ref[...])
    gR = dinv * (aR_ref[0:N, :] + aR_ref[N:, :] + hR_ref[...])
    w = w_ref[...]
    h = jnp.dot(gL, w[:64, :], preferred_element_type=jnp.float32)
    h = h + jnp.dot(gR, w[64:, :], preferred_element_type=jnp.float32)
    o_ref[...] = h + b_ref[0:1, :]


def _f32(*shape):
    return jax.ShapeDtypeStruct(shape, jnp.float32)


_PAD_SRC = (np.arange(PADC * K) % N).astype(np.int32).reshape(PADC, K)
_PAD_DST = (N + np.arange(PADC * K) % (NPAD - N)).astype(
    np.int32).reshape(PADC, K)


def kernel(x, edge_index, W1, b1, W2, b2, W3, b3, W4, b4):
    ei3 = edge_index.astype(jnp.int32).reshape(2, EROWS, K)
    psrc = jnp.asarray(_PAD_SRC)
    pdst = jnp.asarray(_PAD_DST)

    degp = _make_deg()(ei3, pdst)

    lin1n, dinv = pl.pallas_call(
        _tc1_body, out_shape=(_f32(N, 32), _f32(N, 1)))(x, W1, degp)

    a = _make_agg(32)(ei3, psrc, pdst, lin1n)
    h1n = pl.pallas_call(_tc2_body, out_shape=_f32(N, 32))(
        a, lin1n, dinv, jnp.broadcast_to(b1, (8, 32)))

    a = _make_agg(32)(ei3, psrc, pdst, h1n)
    h2n = pl.pallas_call(_tc34_body, out_shape=_f32(N, 64))(
        a, h1n, dinv, W2, jnp.broadcast_to(b2, (8, 64)))

    a = _make_agg(64)(ei3, psrc, pdst, h2n)
    h3nL, h3nR = pl.pallas_call(
        _tc4_body, out_shape=(_f32(N, 64), _f32(N, 64)))(
        a, h2n, dinv, W3, jnp.broadcast_to(b3, (8, 128)))

    aL = _make_agg(64)(ei3, psrc, pdst, h3nL)
    aR = _make_agg(64)(ei3, psrc, pdst, h3nR)
    out = pl.pallas_call(_tc5_body, out_shape=_f32(N, 128))(
        aL, aR, h3nL, h3nR, dinv, W4, jnp.broadcast_to(b4, (8, 128)))
    return out
